# baseline (speedup 1.0000x reference)
import functools

import jax
import jax.numpy as jnp
from jax import lax
from jax.experimental import pallas as pl
from jax.experimental.pallas import tpu as pltpu


def _conv_pool_kernel(x_ref, w_ref, b_ref, pm_ref, featbf_ref, pool_ref, *, bb):
    for i in range(bb):
        feat = jnp.dot(w_ref[...], x_ref[i],
                       preferred_element_type=jnp.float32) + b_ref[...]
        featbf_ref[i] = feat.astype(jnp.bfloat16)
        pool_ref[i] = lax.dot_general(
            feat, pm_ref[...], (((1,), (1,)), ((), ())),
            preferred_element_type=jnp.float32)


def _logit_loss_kernel(pool_ref, emb_ref, lab_ref, pred_ref, loss_ref,
                       *, n_classes, pool_scale):
    pooled = pool_ref[...] * pool_scale
    logits = lax.dot_general(
        pooled, emb_ref[...], (((0,), (0,)), ((), ())),
        preferred_element_type=jnp.float32)
    col = lax.broadcasted_iota(jnp.int32, logits.shape, 1)
    logits = jnp.where(col < n_classes, logits, -1e30)
    m = jnp.max(logits, axis=1, keepdims=True)
    pred_ref[...] = jnp.min(
        jnp.where(logits == m, col, logits.shape[1]),
        axis=1, keepdims=True).astype(jnp.int32)
    lse = m + jnp.log(jnp.sum(jnp.exp(logits - m), axis=1, keepdims=True))
    correct = jnp.sum(jnp.where(col == lab_ref[...], logits, 0.0),
                      axis=1, keepdims=True)
    loss_ref[...] = jnp.sum(lse - correct).reshape(1, 1)


def kernel(latent, labels, emb_dict, conv_w, conv_b):
    B, Cin, H, W = latent.shape
    Cemb = conv_w.shape[0]
    P = 2
    PP = P * P
    HW = H * W
    hb, wb = H // P, W // P
    K = int(emb_dict.shape[0])
    K_pad = max(128, ((K + 127) // 128) * 128)
    R = B * PP
    BB = 4

    x3 = latent.reshape(B, Cin, HW)
    w_mat = conv_w.reshape(Cemb, Cin).astype(jnp.float32)
    b_col = conv_b.reshape(Cemb, 1).astype(jnp.float32)

    hi = jnp.arange(H) // hb
    wi = jnp.arange(W) // wb
    pid = (hi[:, None] * P + wi[None, :]).reshape(HW)
    pm = jax.nn.one_hot(pid, PP, dtype=jnp.float32).T

    emb_t = jnp.zeros((Cemb, K_pad), jnp.float32).at[:, :K].set(
        emb_dict.astype(jnp.float32).T)
    labels_rep = jnp.repeat(labels.astype(jnp.int32), PP).reshape(R, 1)

    featbf, poolf = pl.pallas_call(
        functools.partial(_conv_pool_kernel, bb=BB),
        grid=(B // BB,),
        in_specs=[
            pl.BlockSpec((BB, Cin, HW), lambda b: (b, 0, 0)),
            pl.BlockSpec((Cemb, Cin), lambda b: (0, 0)),
            pl.BlockSpec((Cemb, 1), lambda b: (0, 0)),
            pl.BlockSpec((PP, HW), lambda b: (0, 0)),
        ],
        out_specs=[
            pl.BlockSpec((BB, Cemb, HW), lambda b: (b, 0, 0)),
            pl.BlockSpec((BB, Cemb, PP), lambda b: (b, 0, 0)),
        ],
        out_shape=[
            jax.ShapeDtypeStruct((B, Cemb, HW), jnp.bfloat16),
            jax.ShapeDtypeStruct((B, Cemb, PP), jnp.float32),
        ],
        compiler_params=pltpu.CompilerParams(
            dimension_semantics=("parallel",)),
        cost_estimate=pl.CostEstimate(
            flops=int(2 * B * HW * Cin * Cemb + 2 * B * HW * Cemb * PP),
            transcendentals=0,
            bytes_accessed=int(B * HW * Cin * 4 + B * HW * Cemb * 2
                               + (Cemb * Cin + Cemb + PP * HW) * 4
                               + B * PP * Cemb * 4)),
    )(x3, w_mat, b_col, pm)

    pool_cols = poolf.transpose(1, 0, 2).reshape(Cemb, R)

    pred2, loss2 = pl.pallas_call(
        functools.partial(_logit_loss_kernel, n_classes=K,
                          pool_scale=1.0 / float(hb * wb)),
        grid=(1,),
        in_specs=[
            pl.BlockSpec((Cemb, R), lambda i: (0, 0)),
            pl.BlockSpec((Cemb, K_pad), lambda i: (0, 0)),
            pl.BlockSpec((R, 1), lambda i: (0, 0)),
        ],
        out_specs=[
            pl.BlockSpec((R, 1), lambda i: (0, 0)),
            pl.BlockSpec((1, 1), lambda i: (0, 0)),
        ],
        out_shape=[
            jax.ShapeDtypeStruct((R, 1), jnp.int32),
            jax.ShapeDtypeStruct((1, 1), jnp.float32),
        ],
        compiler_params=pltpu.CompilerParams(
            dimension_semantics=("arbitrary",)),
        cost_estimate=pl.CostEstimate(
            flops=int(2 * R * Cemb * K_pad),
            transcendentals=int(R * K_pad),
            bytes_accessed=int((Cemb * R + Cemb * K_pad) * 4 + R * 8)),
    )(pool_cols, emb_t, labels_rep)

    x_out = featbf.astype(jnp.float32).reshape(B, Cemb, H, W)
    pred = pred2.reshape(R)
    loss = loss2[0, 0] / float(R)
    return x_out, pred, loss, {'dict_loss': loss}


# speedup vs baseline: 1.0294x; 1.0294x over previous
import functools

import jax
import jax.numpy as jnp
from jax import lax
from jax.experimental import pallas as pl
from jax.experimental.pallas import tpu as pltpu


def _conv_pool_kernel(x_ref, w_ref, b_ref, pm_ref, featbf_ref, pool_ref, *, bb):
    for i in range(bb):
        feat = jnp.dot(w_ref[...], x_ref[i],
                       preferred_element_type=jnp.float32) + b_ref[...]
        featbf_ref[i] = feat.astype(jnp.bfloat16)
        pool_ref[i] = lax.dot_general(
            feat, pm_ref[...], (((1,), (1,)), ((), ())),
            preferred_element_type=jnp.float32)


def _logit_loss_kernel(pool_ref, emb_ref, lab_ref, pred_ref, loss_ref,
                       *, n_classes, pool_scale):
    pooled = pool_ref[...] * pool_scale
    logits = lax.dot_general(
        pooled, emb_ref[...], (((0,), (0,)), ((), ())),
        preferred_element_type=jnp.float32)
    col = lax.broadcasted_iota(jnp.int32, logits.shape, 1)
    logits = jnp.where(col < n_classes, logits, -1e30)
    m = jnp.max(logits, axis=1, keepdims=True)
    pred_ref[...] = jnp.min(
        jnp.where(logits == m, col, logits.shape[1]),
        axis=1, keepdims=True).astype(jnp.int32)
    lse = m + jnp.log(jnp.sum(jnp.exp(logits - m), axis=1, keepdims=True))
    correct = jnp.sum(jnp.where(col == lab_ref[...], logits, 0.0),
                      axis=1, keepdims=True)
    loss_ref[...] = jnp.sum(lse - correct).reshape(1, 1)


def kernel(latent, labels, emb_dict, conv_w, conv_b):
    B, Cin, H, W = latent.shape
    Cemb = conv_w.shape[0]
    P = 2
    PP = P * P
    HW = H * W
    hb, wb = H // P, W // P
    K = int(emb_dict.shape[0])
    K_pad = max(128, ((K + 127) // 128) * 128)
    R = B * PP
    BB = 16 if B % 16 == 0 else (8 if B % 8 == 0 else (4 if B % 4 == 0 else 1))

    x3 = latent.reshape(B, Cin, HW)
    w_mat = conv_w.reshape(Cemb, Cin).astype(jnp.float32)
    b_col = conv_b.reshape(Cemb, 1).astype(jnp.float32)

    hi = jnp.arange(H) // hb
    wi = jnp.arange(W) // wb
    pid = (hi[:, None] * P + wi[None, :]).reshape(HW)
    pm = jax.nn.one_hot(pid, PP, dtype=jnp.float32).T

    emb_t = jnp.zeros((Cemb, K_pad), jnp.float32).at[:, :K].set(
        emb_dict.astype(jnp.float32).T)
    labels_rep = jnp.repeat(labels.astype(jnp.int32), PP).reshape(R, 1)

    featbf, poolf = pl.pallas_call(
        functools.partial(_conv_pool_kernel, bb=BB),
        grid=(B // BB,),
        in_specs=[
            pl.BlockSpec((BB, Cin, HW), lambda b: (b, 0, 0)),
            pl.BlockSpec((Cemb, Cin), lambda b: (0, 0)),
            pl.BlockSpec((Cemb, 1), lambda b: (0, 0)),
            pl.BlockSpec((PP, HW), lambda b: (0, 0)),
        ],
        out_specs=[
            pl.BlockSpec((BB, Cemb, HW), lambda b: (b, 0, 0)),
            pl.BlockSpec((BB, Cemb, PP), lambda b: (b, 0, 0)),
        ],
        out_shape=[
            jax.ShapeDtypeStruct((B, Cemb, HW), jnp.bfloat16),
            jax.ShapeDtypeStruct((B, Cemb, PP), jnp.float32),
        ],
        compiler_params=pltpu.CompilerParams(
            dimension_semantics=("parallel",),
            vmem_limit_bytes=int(0.92 * (64 << 20))),
    )(x3, w_mat, b_col, pm)

    pool_cols = poolf.transpose(1, 0, 2).reshape(Cemb, R)

    pred2, loss2 = pl.pallas_call(
        functools.partial(_logit_loss_kernel, n_classes=K,
                          pool_scale=1.0 / float(hb * wb)),
        grid=(1,),
        in_specs=[
            pl.BlockSpec((Cemb, R), lambda i: (0, 0)),
            pl.BlockSpec((Cemb, K_pad), lambda i: (0, 0)),
            pl.BlockSpec((R, 1), lambda i: (0, 0)),
        ],
        out_specs=[
            pl.BlockSpec((R, 1), lambda i: (0, 0)),
            pl.BlockSpec((1, 1), lambda i: (0, 0)),
        ],
        out_shape=[
            jax.ShapeDtypeStruct((R, 1), jnp.int32),
            jax.ShapeDtypeStruct((1, 1), jnp.float32),
        ],
        compiler_params=pltpu.CompilerParams(
            dimension_semantics=("arbitrary",)),
        cost_estimate=pl.CostEstimate(
            flops=int(2 * R * Cemb * K_pad),
            transcendentals=int(R * K_pad),
            bytes_accessed=int((Cemb * R + Cemb * K_pad) * 4 + R * 8)),
    )(pool_cols, emb_t, labels_rep)

    x_out = featbf.reshape(B, Cemb, H, W).astype(jnp.float32)
    pred = pred2.reshape(R)
    loss = loss2[0, 0] / float(R)
    return x_out, pred, loss, {'dict_loss': loss}
